# Initial kernel scaffold; baseline (speedup 1.0000x reference)
import jax
import jax.numpy as jnp
from jax.experimental import pallas as pl


def kernel(src_canonical_coords, src_descriptors, src_uncertainty, src_matchability_logits, tgt_canonical_coords, tgt_descriptors, tgt_uncertainty, tgt_matchability_logits):
    raise NotImplementedError("write your pallas kernel here")


# speedup vs baseline: 8.5349x; 8.5349x over previous
import jax
import jax.numpy as jnp
from jax.experimental import pallas as pl
from jax.experimental.pallas import tpu as pltpu

_B = 2
_C = 128
_D = _H = _W = 16
_N = _D * _H * _W
_M = _N
_K = 24
_TEMPERATURE = 0.07
_RADIUS = 0.45
_RBLK = 256
_NB = _N // _RBLK


def _tc_body(src_can_ref, src_desc_ref, src_ml_ref,
             tgt_canT_ref, tgt_descT_ref, tgt_ml_ref,
             probs_ref, exppos_ref, conf_ref, margin_ref, entropy_ref):
    sc = src_can_ref[0]
    tcT = tgt_canT_ref[0]
    s2 = jnp.sum(sc * sc, axis=1, keepdims=True)
    t2 = jnp.sum(tcT * tcT, axis=0, keepdims=True)
    dotst = jax.lax.dot(sc, tcT,
                        preferred_element_type=jnp.float32)
    d2 = jnp.maximum(s2 + t2 - 2.0 * dotst, 0.0)

    corr = jax.lax.dot(src_desc_ref[0], tgt_descT_ref[0],
                       preferred_element_type=jnp.float32)

    tml = tgt_ml_ref[0]
    tmatch = jax.nn.sigmoid(tml[0:1, :] - tml[1:2, :])
    sml = src_ml_ref[0]
    smatch = jax.nn.sigmoid(sml[:, 0:1] - sml[:, 1:2])

    iota = jax.lax.broadcasted_iota(jnp.int32, (_RBLK, _M), 1)
    tmatch_b = jnp.broadcast_to(tmatch, (_RBLK, _M))
    work = d2
    ls, ds, idxs, tms = [], [], [], []
    for _ in range(_K):
        m = jnp.min(work, axis=1, keepdims=True)
        cand = jnp.where(work == m, iota, _M)
        idxk = jnp.min(cand, axis=1, keepdims=True)
        sel = iota == idxk
        corrk = jnp.sum(jnp.where(sel, corr, 0.0), axis=1, keepdims=True)
        tmk = jnp.sum(jnp.where(sel, tmatch_b, 0.0), axis=1, keepdims=True)
        work = jnp.where(sel, jnp.float32(jnp.inf), work)
        ls.append(corrk / _TEMPERATURE - m / (_RADIUS * _RADIUS))
        ds.append(m)
        idxs.append(idxk)
        tms.append(tmk)

    logits = jnp.concatenate(ls, axis=1)
    idx = jnp.concatenate(idxs, axis=1)
    tmg = jnp.concatenate(tms, axis=1)

    mx = jnp.max(logits, axis=1, keepdims=True)
    e = jnp.exp(logits - mx)
    s = jnp.sum(e, axis=1, keepdims=True)
    p = e / s
    probs_ref[0] = p

    zf = (idx // (_H * _W)).astype(jnp.float32)
    yf = ((idx // _W) % _H).astype(jnp.float32)
    xf = (idx % _W).astype(jnp.float32)
    ez = jnp.sum(p * zf, axis=1, keepdims=True)
    ey = jnp.sum(p * yf, axis=1, keepdims=True)
    ex = jnp.sum(p * xf, axis=1, keepdims=True)
    exppos_ref[0] = jnp.concatenate([ez, ey, ex], axis=1)

    top1 = jnp.max(p, axis=1, keepdims=True)
    kio = jax.lax.broadcasted_iota(jnp.int32, (_RBLK, _K), 1)
    bidx = jnp.min(jnp.where(p == top1, kio, _K), axis=1, keepdims=True)
    bsel = kio == bidx
    second = jnp.max(jnp.where(bsel, -jnp.inf, p), axis=1, keepdims=True)
    btm = jnp.sum(jnp.where(bsel, tmg, 0.0), axis=1, keepdims=True)
    conf_ref[0] = top1 * jnp.sqrt(jnp.clip(smatch * btm, 1e-6, None))
    margin_ref[0] = top1 - second
    entropy_ref[0] = -jnp.sum(p * jnp.log(jnp.clip(p, 1e-12, None)),
                              axis=1, keepdims=True)


def _run_tc(src_can, src_desc, src_ml, tgt_canT, tgt_descT, tgt_ml):
    grid = (_B, _NB)
    out_shapes = (
        jax.ShapeDtypeStruct((_B, _N, _K), jnp.float32),
        jax.ShapeDtypeStruct((_B, _N, 3), jnp.float32),
        jax.ShapeDtypeStruct((_B, _N, 1), jnp.float32),
        jax.ShapeDtypeStruct((_B, _N, 1), jnp.float32),
        jax.ShapeDtypeStruct((_B, _N, 1), jnp.float32),
    )
    in_specs = [
        pl.BlockSpec((1, _RBLK, 3), lambda b, i: (b, i, 0)),
        pl.BlockSpec((1, _RBLK, _C), lambda b, i: (b, i, 0)),
        pl.BlockSpec((1, _RBLK, 2), lambda b, i: (b, i, 0)),
        pl.BlockSpec((1, 3, _M), lambda b, i: (b, 0, 0)),
        pl.BlockSpec((1, _C, _M), lambda b, i: (b, 0, 0)),
        pl.BlockSpec((1, 2, _M), lambda b, i: (b, 0, 0)),
    ]
    out_specs = (
        pl.BlockSpec((1, _RBLK, _K), lambda b, i: (b, i, 0)),
        pl.BlockSpec((1, _RBLK, 3), lambda b, i: (b, i, 0)),
        pl.BlockSpec((1, _RBLK, 1), lambda b, i: (b, i, 0)),
        pl.BlockSpec((1, _RBLK, 1), lambda b, i: (b, i, 0)),
        pl.BlockSpec((1, _RBLK, 1), lambda b, i: (b, i, 0)),
    )
    return pl.pallas_call(
        _tc_body,
        grid=grid,
        in_specs=in_specs,
        out_specs=out_specs,
        out_shape=out_shapes,
        compiler_params=pltpu.CompilerParams(
            dimension_semantics=("parallel", "parallel")),
    )(src_can, src_desc, src_ml, tgt_canT, tgt_descT, tgt_ml)


def kernel(src_canonical_coords, src_descriptors, src_uncertainty,
           src_matchability_logits, tgt_canonical_coords, tgt_descriptors,
           tgt_uncertainty, tgt_matchability_logits):
    b = src_canonical_coords.shape[0]
    spatial = src_canonical_coords.shape[2:]

    src_can = jnp.transpose(src_canonical_coords.reshape(b, 3, _N), (0, 2, 1))
    src_desc = jnp.transpose(src_descriptors.reshape(b, _C, _N), (0, 2, 1))
    src_ml = jnp.transpose(src_matchability_logits.reshape(b, 2, _N), (0, 2, 1))
    tgt_canT = tgt_canonical_coords.reshape(b, 3, _M)
    tgt_descT = tgt_descriptors.reshape(b, _C, _M)
    tgt_ml = tgt_matchability_logits.reshape(b, 2, _M)

    probs, exp_pos, conf, margin, entropy = _run_tc(
        src_can, src_desc, src_ml, tgt_canT, tgt_descT, tgt_ml)

    zz, yy, xx = jnp.meshgrid(jnp.arange(_D), jnp.arange(_H), jnp.arange(_W),
                              indexing='ij')
    grid_pos = jnp.stack([zz, yy, xx], axis=0).astype(jnp.float32)
    positions = jnp.transpose(
        jnp.broadcast_to(grid_pos.reshape(1, 3, _N), (b, 3, _N)), (0, 2, 1))

    raw_displacement = jnp.transpose(exp_pos - positions, (0, 2, 1))
    raw_displacement = raw_displacement.reshape(b, 3, *spatial)
    raw_displacement = jnp.nan_to_num(raw_displacement, nan=0.0,
                                      posinf=0.0, neginf=0.0)

    confidence = conf.reshape(b, 1, *spatial)
    margin = jnp.nan_to_num(margin, nan=0.0, posinf=0.0, neginf=0.0)
    margin = margin.reshape(b, 1, *spatial)
    entropy = jnp.nan_to_num(entropy, nan=0.0, posinf=0.0, neginf=0.0)
    entropy = entropy.reshape(b, 1, *spatial)
    probs = jnp.nan_to_num(probs, nan=0.0, posinf=0.0, neginf=0.0)

    return (exp_pos, raw_displacement, probs, confidence, margin, entropy,
            positions)


# speedup vs baseline: 13.2407x; 1.5514x over previous
import functools

import jax
import jax.numpy as jnp
from jax import lax
from jax.experimental import pallas as pl
from jax.experimental.pallas import tpu as pltpu
from jax.experimental.pallas import tpu_sc as plsc

_B = 2
_C = 128
_D = _H = _W = 16
_N = _D * _H * _W
_M = _N
_K = 24
_TEMPERATURE = 0.07
_RADIUS = 0.45
_RBLK = 256
_NB = _N // _RBLK

_PAIRS = _B * _N * _K
_NW = 32
_PW = _PAIRS // _NW
_GCH = 128
_NGC = _PW // _GCH
_VCH = 16
_NVC = _PW // _VCH


def _tc_topk_body(src_can_ref, tgt_canT_ref, src_desc_ref, tgt_descT_ref,
                  idx_ref, d2k_ref, corr_ref):
    sc = src_can_ref[0]
    tcT = tgt_canT_ref[0]
    s2 = jnp.sum(sc * sc, axis=1, keepdims=True)
    t2 = jnp.sum(tcT * tcT, axis=0, keepdims=True)
    dotst = jax.lax.dot(sc, tcT,
                        preferred_element_type=jnp.float32)
    d2 = jnp.maximum(s2 + t2 - 2.0 * dotst, 0.0)

    corr_ref[0] = jax.lax.dot(src_desc_ref[0], tgt_descT_ref[0],
                              preferred_element_type=jnp.float32)

    iota = jax.lax.broadcasted_iota(jnp.int32, (_RBLK, _M), 1)
    work = d2
    ds, idxs = [], []
    for _ in range(_K):
        m = jnp.min(work, axis=1, keepdims=True)
        cand = jnp.where(work == m, iota, _M)
        idxk = jnp.min(cand, axis=1, keepdims=True)
        sel = iota == idxk
        work = jnp.where(sel, jnp.float32(jnp.inf), work)
        ds.append(m)
        idxs.append(idxk)

    idx_ref[0] = jnp.concatenate(idxs, axis=1)
    d2k_ref[0] = jnp.concatenate(ds, axis=1)


def _run_tc_topk(src_can, tgt_canT, src_desc, tgt_descT):
    grid = (_B, _NB)
    out_shapes = (
        jax.ShapeDtypeStruct((_B, _N, _K), jnp.int32),
        jax.ShapeDtypeStruct((_B, _N, _K), jnp.float32),
        jax.ShapeDtypeStruct((_B, _N, _M), jnp.float32),
    )
    in_specs = [
        pl.BlockSpec((1, _RBLK, 3), lambda b, i: (b, i, 0)),
        pl.BlockSpec((1, 3, _M), lambda b, i: (b, 0, 0)),
        pl.BlockSpec((1, _RBLK, _C), lambda b, i: (b, i, 0)),
        pl.BlockSpec((1, _C, _M), lambda b, i: (b, 0, 0)),
    ]
    out_specs = (
        pl.BlockSpec((1, _RBLK, _K), lambda b, i: (b, i, 0)),
        pl.BlockSpec((1, _RBLK, _K), lambda b, i: (b, i, 0)),
        pl.BlockSpec((1, _RBLK, _M), lambda b, i: (b, i, 0)),
    )
    return pl.pallas_call(
        _tc_topk_body,
        grid=grid,
        in_specs=in_specs,
        out_specs=out_specs,
        out_shape=out_shapes,
        compiler_params=pltpu.CompilerParams(
            dimension_semantics=("parallel", "parallel")),
    )(src_can, tgt_canT, src_desc, tgt_descT)


def _run_sc_gather(idx_flat, corr_base, ml_base, corr_flat, ml_flat):
    mesh = plsc.VectorSubcoreMesh(core_axis_name="c", subcore_axis_name="s")

    @functools.partial(
        pl.kernel, mesh=mesh,
        out_type=(
            jax.ShapeDtypeStruct((_PAIRS,), jnp.float32),
            jax.ShapeDtypeStruct((_PAIRS,), jnp.float32),
        ),
        scratch_types=[
            pltpu.VMEM((_PW,), jnp.int32),
            pltpu.VMEM((_PW,), jnp.int32),
            pltpu.VMEM((_PW,), jnp.int32),
            pltpu.VMEM((_PW,), jnp.int32),
            pltpu.VMEM((_PW,), jnp.int32),
            pltpu.VMEM((_PW,), jnp.int32),
            pltpu.VMEM((_PW,), jnp.float32),
            pltpu.VMEM((_PW,), jnp.float32),
            pltpu.VMEM((_PW,), jnp.float32),
            pltpu.SemaphoreType.DMA,
        ],
    )
    def k(idx_hbm, cbase_hbm, mbase_hbm, corr_hbm, ml_hbm,
          ock_hbm, otm_hbm,
          idx_v, cb_v, mb_v, ci_v, m0i_v, m1i_v, cv_v, m0v_v, m1v_v, sem):
        wid = lax.axis_index("s") * 2 + lax.axis_index("c")
        base = wid * _PW
        pltpu.sync_copy(idx_hbm.at[pl.ds(base, _PW)], idx_v)
        pltpu.sync_copy(cbase_hbm.at[pl.ds(base, _PW)], cb_v)
        pltpu.sync_copy(mbase_hbm.at[pl.ds(base, _PW)], mb_v)

        def build(j, carry):
            s = pl.ds(j * _VCH, _VCH)
            v = idx_v[s]
            ci_v[s] = v + cb_v[s]
            m0 = v + mb_v[s]
            m0i_v[s] = m0
            m1i_v[s] = m0 + _M
            return carry

        lax.fori_loop(0, _NVC, build, 0)

        def gather(c, carry):
            s = pl.ds(c * _GCH, _GCH)
            pltpu.async_copy(corr_hbm.at[ci_v.at[s]], cv_v.at[s], sem).wait()
            pltpu.async_copy(ml_hbm.at[m0i_v.at[s]], m0v_v.at[s], sem).wait()
            pltpu.async_copy(ml_hbm.at[m1i_v.at[s]], m1v_v.at[s], sem).wait()
            return carry

        lax.fori_loop(0, _NGC, gather, 0)

        def combine(j, carry):
            s = pl.ds(j * _VCH, _VCH)
            m1v_v[s] = 1.0 / (1.0 + jnp.exp(m1v_v[s] - m0v_v[s]))
            return carry

        lax.fori_loop(0, _NVC, combine, 0)

        pltpu.sync_copy(cv_v, ock_hbm.at[pl.ds(base, _PW)])
        pltpu.sync_copy(m1v_v, otm_hbm.at[pl.ds(base, _PW)])

    return k(idx_flat, corr_base, ml_base, corr_flat, ml_flat)


def _tc_combine_body(idx_ref, d2k_ref, corrk_ref, tmk_ref, src_ml_ref,
                     probs_ref, exppos_ref, conf_ref, margin_ref,
                     entropy_ref):
    idx = idx_ref[0]
    d2 = d2k_ref[0]
    logits = corrk_ref[0] / _TEMPERATURE - d2 / (_RADIUS * _RADIUS)
    tmg = tmk_ref[0]
    sml = src_ml_ref[0]
    smatch = jax.nn.sigmoid(sml[:, 0:1] - sml[:, 1:2])

    mx = jnp.max(logits, axis=1, keepdims=True)
    e = jnp.exp(logits - mx)
    s = jnp.sum(e, axis=1, keepdims=True)
    p = e / s
    probs_ref[0] = p

    zf = (idx // (_H * _W)).astype(jnp.float32)
    yf = ((idx // _W) % _H).astype(jnp.float32)
    xf = (idx % _W).astype(jnp.float32)
    ez = jnp.sum(p * zf, axis=1, keepdims=True)
    ey = jnp.sum(p * yf, axis=1, keepdims=True)
    ex = jnp.sum(p * xf, axis=1, keepdims=True)
    exppos_ref[0] = jnp.concatenate([ez, ey, ex], axis=1)

    top1 = jnp.max(p, axis=1, keepdims=True)
    kio = jax.lax.broadcasted_iota(jnp.int32, (_RBLK, _K), 1)
    bidx = jnp.min(jnp.where(p == top1, kio, _K), axis=1, keepdims=True)
    bsel = kio == bidx
    second = jnp.max(jnp.where(bsel, -jnp.inf, p), axis=1, keepdims=True)
    btm = jnp.sum(jnp.where(bsel, tmg, 0.0), axis=1, keepdims=True)
    conf_ref[0] = top1 * jnp.sqrt(jnp.clip(smatch * btm, 1e-6, None))
    margin_ref[0] = top1 - second
    entropy_ref[0] = -jnp.sum(p * jnp.log(jnp.clip(p, 1e-12, None)),
                              axis=1, keepdims=True)


def _run_tc_combine(idx, d2k, corr_k, tm_k, src_ml):
    grid = (_B, _NB)
    out_shapes = (
        jax.ShapeDtypeStruct((_B, _N, _K), jnp.float32),
        jax.ShapeDtypeStruct((_B, _N, 3), jnp.float32),
        jax.ShapeDtypeStruct((_B, _N, 1), jnp.float32),
        jax.ShapeDtypeStruct((_B, _N, 1), jnp.float32),
        jax.ShapeDtypeStruct((_B, _N, 1), jnp.float32),
    )
    in_specs = [
        pl.BlockSpec((1, _RBLK, _K), lambda b, i: (b, i, 0)),
        pl.BlockSpec((1, _RBLK, _K), lambda b, i: (b, i, 0)),
        pl.BlockSpec((1, _RBLK, _K), lambda b, i: (b, i, 0)),
        pl.BlockSpec((1, _RBLK, _K), lambda b, i: (b, i, 0)),
        pl.BlockSpec((1, _RBLK, 2), lambda b, i: (b, i, 0)),
    ]
    out_specs = (
        pl.BlockSpec((1, _RBLK, _K), lambda b, i: (b, i, 0)),
        pl.BlockSpec((1, _RBLK, 3), lambda b, i: (b, i, 0)),
        pl.BlockSpec((1, _RBLK, 1), lambda b, i: (b, i, 0)),
        pl.BlockSpec((1, _RBLK, 1), lambda b, i: (b, i, 0)),
        pl.BlockSpec((1, _RBLK, 1), lambda b, i: (b, i, 0)),
    )
    return pl.pallas_call(
        _tc_combine_body,
        grid=grid,
        in_specs=in_specs,
        out_specs=out_specs,
        out_shape=out_shapes,
        compiler_params=pltpu.CompilerParams(
            dimension_semantics=("parallel", "parallel")),
    )(idx, d2k, corr_k, tm_k, src_ml)


def kernel(src_canonical_coords, src_descriptors, src_uncertainty,
           src_matchability_logits, tgt_canonical_coords, tgt_descriptors,
           tgt_uncertainty, tgt_matchability_logits):
    b = src_canonical_coords.shape[0]
    spatial = src_canonical_coords.shape[2:]

    src_can = jnp.transpose(src_canonical_coords.reshape(b, 3, _N), (0, 2, 1))
    src_desc = jnp.transpose(src_descriptors.reshape(b, _C, _N), (0, 2, 1))
    src_ml = jnp.transpose(src_matchability_logits.reshape(b, 2, _N), (0, 2, 1))
    tgt_canT = tgt_canonical_coords.reshape(b, 3, _M)
    tgt_descT = tgt_descriptors.reshape(b, _C, _M)
    tgt_ml = tgt_matchability_logits.reshape(b, 2, _M)

    idx, d2k, corr = _run_tc_topk(src_can, tgt_canT, src_desc, tgt_descT)

    pair = jnp.arange(_PAIRS, dtype=jnp.int32)
    corr_base = (pair // _K) * _M
    ml_base = (pair // (_N * _K)) * (2 * _M)
    corr_k_flat, tm_k_flat = _run_sc_gather(
        idx.reshape(_PAIRS), corr_base, ml_base,
        corr.reshape(_B * _N * _M), tgt_ml.reshape(_B * 2 * _M))
    corr_k = corr_k_flat.reshape(_B, _N, _K)
    tm_k = tm_k_flat.reshape(_B, _N, _K)

    probs, exp_pos, conf, margin, entropy = _run_tc_combine(
        idx, d2k, corr_k, tm_k, src_ml)

    zz, yy, xx = jnp.meshgrid(jnp.arange(_D), jnp.arange(_H), jnp.arange(_W),
                              indexing='ij')
    grid_pos = jnp.stack([zz, yy, xx], axis=0).astype(jnp.float32)
    positions = jnp.transpose(
        jnp.broadcast_to(grid_pos.reshape(1, 3, _N), (b, 3, _N)), (0, 2, 1))

    raw_displacement = jnp.transpose(exp_pos - positions, (0, 2, 1))
    raw_displacement = raw_displacement.reshape(b, 3, *spatial)
    raw_displacement = jnp.nan_to_num(raw_displacement, nan=0.0,
                                      posinf=0.0, neginf=0.0)

    confidence = conf.reshape(b, 1, *spatial)
    margin = jnp.nan_to_num(margin, nan=0.0, posinf=0.0, neginf=0.0)
    margin = margin.reshape(b, 1, *spatial)
    entropy = jnp.nan_to_num(entropy, nan=0.0, posinf=0.0, neginf=0.0)
    entropy = entropy.reshape(b, 1, *spatial)
    probs = jnp.nan_to_num(probs, nan=0.0, posinf=0.0, neginf=0.0)

    return (exp_pos, raw_displacement, probs, confidence, margin, entropy,
            positions)
